# baseline (speedup 1.0000x reference)
import functools

import jax
import jax.numpy as jnp
from jax import lax
from jax.experimental import pallas as pl
from jax.experimental.pallas import tpu as pltpu
from jax.experimental.pallas import tpu_sc as plsc

N = 10000
E = 320000
D = 128
OUT = 128
L = 3
EPS = 1e-7

NS = 16
CH = 64
E_PAD = 327680
T = E_PAD // (NS * CH)
NB = 4
N_PAD = 10240
ROWS_PT = N_PAD // NS


def _bn_prep(hv, gamma, beta):
    mu = jnp.mean(hv, axis=0, keepdims=True)
    var = jnp.mean((hv - mu) ** 2, axis=0, keepdims=True)
    hv1 = jnp.maximum((hv - mu) * lax.rsqrt(var + 1e-5) * gamma + beta, 0.0)
    p = hv1 + EPS
    m = jnp.max(p, axis=0, keepdims=True)
    q = jnp.exp(p - m)
    return hv1, q, q * p


def _enc_body(x_ref, w_ref, b_ref, g_ref, be_ref,
              hv_ref, hv1_ref, q_ref, r_ref):
    hv = lax.dot_general(x_ref[...], w_ref[...], (((1,), (1,)), ((), ())),
                         preferred_element_type=jnp.float32) + b_ref[...]
    hv1, q, r = _bn_prep(hv, g_ref[...], be_ref[...])
    hv_ref[...] = hv
    hv1_ref[...] = hv1
    q_ref[...] = q
    r_ref[...] = r


def _mid_body(hv_ref, hv1_ref, den_ref, num_ref, w_ref, b_ref, g_ref, be_ref,
              hvn_ref, hv1n_ref, q_ref, r_ref):
    agg = num_ref[:N] / (den_ref[:N] + 1e-16)
    hv = lax.dot_general(hv1_ref[...] + agg, w_ref[...],
                         (((1,), (1,)), ((), ())),
                         preferred_element_type=jnp.float32)
    hv = hv + b_ref[...] + hv_ref[...]
    hv1, q, r = _bn_prep(hv, g_ref[...], be_ref[...])
    hvn_ref[...] = hv
    hv1n_ref[...] = hv1
    q_ref[...] = q
    r_ref[...] = r


def _fin_body(hv_ref, hv1_ref, den_ref, num_ref, w_ref, b_ref,
              wo_ref, bo_ref, out_ref):
    agg = num_ref[:N] / (den_ref[:N] + 1e-16)
    hv = lax.dot_general(hv1_ref[...] + agg, w_ref[...],
                         (((1,), (1,)), ((), ())),
                         preferred_element_type=jnp.float32)
    hv = hv + b_ref[...] + hv_ref[...]
    out_ref[...] = lax.dot_general(hv, wo_ref[...], (((1,), (0,)), ((), ())),
                                   preferred_element_type=jnp.float32) + bo_ref[...]


_ND = jax.ShapeDtypeStruct((N, D), jnp.float32)

_enc_call = pl.pallas_call(_enc_body, out_shape=(_ND, _ND, _ND, _ND))
_mid_call = pl.pallas_call(_mid_body, out_shape=(_ND, _ND, _ND, _ND))
_fin_call = pl.pallas_call(
    _fin_body, out_shape=jax.ShapeDtypeStruct((N, OUT), jnp.float32))


def _edge_body(src_hbm, dst_hbm, q_hbm, r_hbm, den_out, num_out,
               sidx, didx, rows, acc, isem, gsem, ssem):
    c = lax.axis_index("c")
    s = lax.axis_index("s")
    rbase = s * ROWS_PT

    ebase = s * T * CH

    def fire_idx(i, b):
        base = ebase + i * CH
        pltpu.async_copy(src_hbm.at[pl.ds(base, CH)], sidx.at[b], isem.at[b])
        pltpu.async_copy(dst_hbm.at[pl.ds(base, CH)], didx.at[b], isem.at[b])

    def wait_idx(b):
        pltpu.make_async_copy(src_hbm.at[pl.ds(0, CH)], sidx.at[b],
                              isem.at[b]).wait()
        pltpu.make_async_copy(dst_hbm.at[pl.ds(0, CH)], didx.at[b],
                              isem.at[b]).wait()

    def fire_gather(b):
        @pl.when(c == 0)
        def _():
            pltpu.async_copy(q_hbm.at[sidx.at[b]], rows.at[b], gsem.at[b])

        @pl.when(c == 1)
        def _():
            pltpu.async_copy(r_hbm.at[sidx.at[b]], rows.at[b], gsem.at[b])

    def wait_gather(b):
        pltpu.make_async_copy(q_hbm.at[sidx.at[b]], rows.at[b],
                              gsem.at[b]).wait()

    def fire_scatter(b):
        pltpu.async_copy(rows.at[b], acc.at[didx.at[b]], ssem.at[b],
                         add=True)

    def wait_scatter(b):
        pltpu.make_async_copy(rows.at[b], acc.at[didx.at[b]],
                              ssem.at[b]).wait()

    for i in range(NB):
        fire_idx(i, i)
    for i in range(NB - 1):
        wait_idx(i)
        fire_gather(i)

    z16 = jnp.zeros((16,), jnp.float32)

    def zrow(i, carry):
        for j in range(D // 16):
            rows[NB - 1, i, pl.ds(j * 16, 16)] = z16
        return carry

    lax.fori_loop(0, CH, zrow, 0)
    for w in range(ROWS_PT // CH):
        pltpu.sync_copy(rows.at[NB - 1], acc.at[pl.ds(rbase + w * CH, CH)])
    plsc.subcore_barrier()
    for i in range(NB - 2):
        wait_gather(i)
        fire_scatter(i)

    def step(k, carry):
        for b in range(NB):
            i = k * NB + b
            wait_scatter(b)
            fire_idx(i, b)
            wait_idx((b - 1) % NB)
            fire_gather((b - 1) % NB)
            wait_gather((b - 2) % NB)
            fire_scatter((b - 2) % NB)
        return carry

    lax.fori_loop(1, T // NB, step, 0)

    wait_idx((T - 1) % NB)
    fire_gather((T - 1) % NB)
    wait_gather((T - 2) % NB)
    fire_scatter((T - 2) % NB)
    wait_gather((T - 1) % NB)
    fire_scatter((T - 1) % NB)
    for j in range(T - NB, T):
        wait_scatter(j % NB)
    plsc.subcore_barrier()

    sl = pl.ds(rbase, ROWS_PT)

    @pl.when(c == 0)
    def _():
        pltpu.sync_copy(acc.at[sl], den_out.at[sl])

    @pl.when(c == 1)
    def _():
        pltpu.sync_copy(acc.at[sl], num_out.at[sl])


_NPD = jax.ShapeDtypeStruct((N_PAD, D), jnp.float32)

_edge_call = functools.partial(
    pl.kernel,
    out_type=(_NPD, _NPD),
    mesh=plsc.VectorSubcoreMesh(core_axis_name="c", subcore_axis_name="s"),
    scratch_types=[
        pltpu.VMEM((NB, CH), jnp.int32),
        pltpu.VMEM((NB, CH), jnp.int32),
        pltpu.VMEM((NB, CH, D), jnp.float32),
        pltpu.VMEM_SHARED((N_PAD, D), jnp.float32),
        pltpu.SemaphoreType.DMA((NB,)),
        pltpu.SemaphoreType.DMA((NB,)),
        pltpu.SemaphoreType.DMA((NB,)),
    ],
)(_edge_body)


def kernel(x, adj, W_enc, b_enc, bn_gamma, bn_beta, W_gcn, b_gcn, W_out, b_out):
    npad = E_PAD - E
    pad_src = (jnp.arange(npad, dtype=jnp.int32) * 997) % N
    pad_dst = N + (jnp.arange(npad, dtype=jnp.int32) % (N_PAD - N))
    adj_p = jnp.concatenate([adj, jnp.stack([pad_src, pad_dst])], axis=1)
    src_p = adj_p[0]
    dst_p = adj_p[1]
    row = lambda v: v.reshape(1, -1)
    w_fold = 0.5 * jnp.repeat(W_out.T, 2, axis=0)

    hv, hv1, q, r = _enc_call(x, W_enc, row(b_enc),
                              row(bn_gamma[0]), row(bn_beta[0]))
    for l in range(L - 1):
        den, num = _edge_call(src_p, dst_p, q, r)
        hv, hv1, q, r = _mid_call(hv, hv1, den, num, W_gcn[l],
                                  row(b_gcn[l]),
                                  row(bn_gamma[l + 1]), row(bn_beta[l + 1]))
    den, num = _edge_call(src_p, dst_p, q, r)
    return _fin_call(hv, hv1, den, num, W_gcn[L - 1], row(b_gcn[L - 1]),
                     w_fold, row(b_out))


# speedup vs baseline: 1.0006x; 1.0006x over previous
import functools

import jax
import jax.numpy as jnp
from jax import lax
from jax.experimental import pallas as pl
from jax.experimental.pallas import tpu as pltpu
from jax.experimental.pallas import tpu_sc as plsc

N = 10000
E = 320000
D = 128
OUT = 128
L = 3
EPS = 1e-7

NS = 16
CH = 64
E_PAD = 327680
T = E_PAD // (NS * CH)
NB = 5
N_PAD = 10240
ROWS_PT = N_PAD // NS


def _bn_prep(hv, gamma, beta):
    mu = jnp.mean(hv, axis=0, keepdims=True)
    var = jnp.mean((hv - mu) ** 2, axis=0, keepdims=True)
    hv1 = jnp.maximum((hv - mu) * lax.rsqrt(var + 1e-5) * gamma + beta, 0.0)
    p = hv1 + EPS
    m = jnp.max(p, axis=0, keepdims=True)
    q = jnp.exp(p - m)
    return hv1, q, q * p


def _enc_body(x_ref, w_ref, b_ref, g_ref, be_ref,
              hv_ref, hv1_ref, q_ref, r_ref):
    hv = lax.dot_general(x_ref[...], w_ref[...], (((1,), (1,)), ((), ())),
                         preferred_element_type=jnp.float32) + b_ref[...]
    hv1, q, r = _bn_prep(hv, g_ref[...], be_ref[...])
    hv_ref[...] = hv
    hv1_ref[...] = hv1
    q_ref[...] = q
    r_ref[...] = r


def _mid_body(hv_ref, hv1_ref, den_ref, num_ref, w_ref, b_ref, g_ref, be_ref,
              hvn_ref, hv1n_ref, q_ref, r_ref):
    agg = num_ref[:N] / (den_ref[:N] + 1e-16)
    hv = lax.dot_general(hv1_ref[...] + agg, w_ref[...],
                         (((1,), (1,)), ((), ())),
                         preferred_element_type=jnp.float32)
    hv = hv + b_ref[...] + hv_ref[...]
    hv1, q, r = _bn_prep(hv, g_ref[...], be_ref[...])
    hvn_ref[...] = hv
    hv1n_ref[...] = hv1
    q_ref[...] = q
    r_ref[...] = r


def _fin_body(hv_ref, hv1_ref, den_ref, num_ref, w_ref, b_ref,
              wo_ref, bo_ref, out_ref):
    agg = num_ref[:N] / (den_ref[:N] + 1e-16)
    hv = lax.dot_general(hv1_ref[...] + agg, w_ref[...],
                         (((1,), (1,)), ((), ())),
                         preferred_element_type=jnp.float32)
    hv = hv + b_ref[...] + hv_ref[...]
    out_ref[...] = lax.dot_general(hv, wo_ref[...], (((1,), (0,)), ((), ())),
                                   preferred_element_type=jnp.float32) + bo_ref[...]


_ND = jax.ShapeDtypeStruct((N, D), jnp.float32)

_enc_call = pl.pallas_call(_enc_body, out_shape=(_ND, _ND, _ND, _ND))
_mid_call = pl.pallas_call(_mid_body, out_shape=(_ND, _ND, _ND, _ND))
_fin_call = pl.pallas_call(
    _fin_body, out_shape=jax.ShapeDtypeStruct((N, OUT), jnp.float32))


def _edge_body(src_hbm, dst_hbm, q_hbm, r_hbm, den_out, num_out,
               sidx, didx, rows, acc, isem, gsem, ssem):
    c = lax.axis_index("c")
    s = lax.axis_index("s")
    rbase = s * ROWS_PT

    ebase = s * T * CH

    def fire_idx(i, b):
        base = ebase + i * CH
        pltpu.async_copy(src_hbm.at[pl.ds(base, CH)], sidx.at[b], isem.at[b])
        pltpu.async_copy(dst_hbm.at[pl.ds(base, CH)], didx.at[b], isem.at[b])

    def wait_idx(b):
        pltpu.make_async_copy(src_hbm.at[pl.ds(0, CH)], sidx.at[b],
                              isem.at[b]).wait()
        pltpu.make_async_copy(dst_hbm.at[pl.ds(0, CH)], didx.at[b],
                              isem.at[b]).wait()

    def fire_gather(b):
        @pl.when(c == 0)
        def _():
            pltpu.async_copy(q_hbm.at[sidx.at[b]], rows.at[b], gsem.at[b])

        @pl.when(c == 1)
        def _():
            pltpu.async_copy(r_hbm.at[sidx.at[b]], rows.at[b], gsem.at[b])

    def wait_gather(b):
        pltpu.make_async_copy(q_hbm.at[sidx.at[b]], rows.at[b],
                              gsem.at[b]).wait()

    def fire_scatter(b):
        pltpu.async_copy(rows.at[b], acc.at[didx.at[b]], ssem.at[b],
                         add=True)

    def wait_scatter(b):
        pltpu.make_async_copy(rows.at[b], acc.at[didx.at[b]],
                              ssem.at[b]).wait()

    for i in range(NB):
        fire_idx(i, i)
    for i in range(NB - 1):
        wait_idx(i)
        fire_gather(i)

    z16 = jnp.zeros((16,), jnp.float32)

    def zrow(i, carry):
        for j in range(D // 16):
            rows[NB - 1, i, pl.ds(j * 16, 16)] = z16
        return carry

    lax.fori_loop(0, CH, zrow, 0)
    for w in range(ROWS_PT // CH):
        pltpu.sync_copy(rows.at[NB - 1], acc.at[pl.ds(rbase + w * CH, CH)])
    plsc.subcore_barrier()
    for i in range(NB - 2):
        wait_gather(i)
        fire_scatter(i)

    def step(k, carry):
        for b in range(NB):
            i = k * NB + b
            wait_scatter(b)
            fire_idx(i, b)
            wait_idx((b - 1) % NB)
            fire_gather((b - 1) % NB)
            wait_gather((b - 2) % NB)
            fire_scatter((b - 2) % NB)
        return carry

    lax.fori_loop(1, T // NB, step, 0)

    wait_idx((T - 1) % NB)
    fire_gather((T - 1) % NB)
    wait_gather((T - 2) % NB)
    fire_scatter((T - 2) % NB)
    wait_gather((T - 1) % NB)
    fire_scatter((T - 1) % NB)
    for j in range(T - NB, T):
        wait_scatter(j % NB)
    plsc.subcore_barrier()

    sl = pl.ds(rbase, ROWS_PT)

    @pl.when(c == 0)
    def _():
        pltpu.sync_copy(acc.at[sl], den_out.at[sl])

    @pl.when(c == 1)
    def _():
        pltpu.sync_copy(acc.at[sl], num_out.at[sl])


_NPD = jax.ShapeDtypeStruct((N_PAD, D), jnp.float32)

_edge_call = functools.partial(
    pl.kernel,
    out_type=(_NPD, _NPD),
    mesh=plsc.VectorSubcoreMesh(core_axis_name="c", subcore_axis_name="s"),
    scratch_types=[
        pltpu.VMEM((NB, CH), jnp.int32),
        pltpu.VMEM((NB, CH), jnp.int32),
        pltpu.VMEM((NB, CH, D), jnp.float32),
        pltpu.VMEM_SHARED((N_PAD, D), jnp.float32),
        pltpu.SemaphoreType.DMA((NB,)),
        pltpu.SemaphoreType.DMA((NB,)),
        pltpu.SemaphoreType.DMA((NB,)),
    ],
)(_edge_body)


def kernel(x, adj, W_enc, b_enc, bn_gamma, bn_beta, W_gcn, b_gcn, W_out, b_out):
    npad = E_PAD - E
    pad_src = (jnp.arange(npad, dtype=jnp.int32) * 997) % N
    pad_dst = N + (jnp.arange(npad, dtype=jnp.int32) % (N_PAD - N))
    adj_p = jnp.concatenate([adj, jnp.stack([pad_src, pad_dst])], axis=1)
    src_p = adj_p[0]
    dst_p = adj_p[1]
    row = lambda v: v.reshape(1, -1)
    w_fold = 0.5 * jnp.repeat(W_out.T, 2, axis=0)

    hv, hv1, q, r = _enc_call(x, W_enc, row(b_enc),
                              row(bn_gamma[0]), row(bn_beta[0]))
    for l in range(L - 1):
        den, num = _edge_call(src_p, dst_p, q, r)
        hv, hv1, q, r = _mid_call(hv, hv1, den, num, W_gcn[l],
                                  row(b_gcn[l]),
                                  row(bn_gamma[l + 1]), row(bn_beta[l + 1]))
    den, num = _edge_call(src_p, dst_p, q, r)
    return _fin_call(hv, hv1, den, num, W_gcn[L - 1], row(b_gcn[L - 1]),
                     w_fold, row(b_out))
